# baseline (speedup 1.0000x reference)
import jax
import jax.numpy as jnp
from jax import lax
from jax.experimental import pallas as pl
from jax.experimental.pallas import tpu as pltpu
from jax.experimental.pallas import tpu_sc as plsc
import functools

_K = 32
_TR = _K // 8
_LANES = 128
_T = 4
_SC_STEPS = 20
_NW = 32
_NSC = _NW * _SC_STEPS * _T * _LANES


def _tc_body(u_ref, i_ref, o_ref):
    o_ref[...] = jnp.sum(u_ref[...] * i_ref[...], axis=0)


def _sc_body(tc0, u_hbm, i_hbm, o_hbm, u_v, i_v, o_v):
    cid = lax.axis_index("c")
    sid = lax.axis_index("s")
    wid = sid * 2 + cid
    tc_w = tc0 + wid * (_SC_STEPS * _T)

    def step(si, carry):
        tcol = tc_w + si * _T
        pltpu.sync_copy(u_hbm.at[:, pl.ds(tcol, _T), :, :], u_v)
        pltpu.sync_copy(i_hbm.at[:, pl.ds(tcol, _T), :, :], i_v)
        for t in range(_T):
            for l in range(_LANES // 16):
                acc = (u_v[0, t, 0, pl.ds(l * 16, 16)]
                       * i_v[0, t, 0, pl.ds(l * 16, 16)])
                for tr in range(_TR):
                    for r in range(8):
                        if tr == 0 and r == 0:
                            continue
                        acc = acc + (u_v[tr, t, r, pl.ds(l * 16, 16)]
                                     * i_v[tr, t, r, pl.ds(l * 16, 16)])
                o_v[pl.ds(t * _LANES + l * 16, 16)] = acc
        out_off = (wid * _SC_STEPS + si) * (_T * _LANES)
        pltpu.sync_copy(o_v, o_hbm.at[pl.ds(out_off, _T * _LANES)])
        return carry

    lax.fori_loop(0, _SC_STEPS, step, 0)


def kernel(gu, gi):
    gu = jnp.squeeze(gu)
    gi = jnp.squeeze(gi)
    n, k = gu.shape
    ut = gu.T
    it = gi.T
    nsc = _NSC
    ntc = n - nsc

    bn = 32768
    grid = pl.cdiv(ntc, bn)
    tc_out = pl.pallas_call(
        _tc_body,
        grid=(grid,),
        in_specs=[
            pl.BlockSpec((k, bn), lambda i: (0, i)),
            pl.BlockSpec((k, bn), lambda i: (0, i)),
        ],
        out_specs=pl.BlockSpec((bn,), lambda i: (i,)),
        out_shape=jax.ShapeDtypeStruct((ntc,), jnp.float32),
    )(ut, it)

    tcols = n // _LANES
    vt_u = ut.reshape(_TR, 8, tcols, _LANES).transpose(0, 2, 1, 3)
    vt_i = it.reshape(_TR, 8, tcols, _LANES).transpose(0, 2, 1, 3)
    sc_tc0 = ntc // _LANES

    mesh = plsc.VectorSubcoreMesh(core_axis_name="c", subcore_axis_name="s")
    sc_out = pl.kernel(
        functools.partial(_sc_body, sc_tc0),
        out_type=jax.ShapeDtypeStruct((nsc,), jnp.float32),
        mesh=mesh,
        scratch_types=[
            pltpu.VMEM((_TR, _T, 8, _LANES), jnp.float32),
            pltpu.VMEM((_TR, _T, 8, _LANES), jnp.float32),
            pltpu.VMEM((_T * _LANES,), jnp.float32),
        ],
    )(vt_u, vt_i)

    return jnp.concatenate([tc_out, sc_out])


# speedup vs baseline: 1.4781x; 1.4781x over previous
import jax
import jax.numpy as jnp
from jax import lax
from jax.experimental import pallas as pl
from jax.experimental.pallas import tpu as pltpu
from jax.experimental.pallas import tpu_sc as plsc
import functools

_K = 32
_TR = _K // 8
_LANES = 128
_T = 2
_SC_STEPS = 40
_NW = 32
_NSC = _NW * _SC_STEPS * _T * _LANES


def _tc_body(u_ref, i_ref, o_ref):
    o_ref[...] = jnp.sum(u_ref[...] * i_ref[...], axis=0)


def _sc_body(tc0, u_hbm, i_hbm, o_hbm, u_v, i_v, o_v, sem_u, sem_i, sem_o):
    cid = lax.axis_index("c")
    sid = lax.axis_index("s")
    wid = sid * 2 + cid
    tc_w = tc0 + wid * (_SC_STEPS * _T)
    out_w = wid * _SC_STEPS * _T * _LANES

    def issue_in(si, b):
        tcol = tc_w + si * _T
        pltpu.async_copy(u_hbm.at[:, pl.ds(tcol, _T), :, :], u_v.at[b],
                         sem_u.at[b])
        pltpu.async_copy(i_hbm.at[:, pl.ds(tcol, _T), :, :], i_v.at[b],
                         sem_i.at[b])

    def wait_in(b):
        pltpu.make_async_copy(u_hbm.at[:, pl.ds(0, _T), :, :], u_v.at[b],
                              sem_u.at[b]).wait()
        pltpu.make_async_copy(i_hbm.at[:, pl.ds(0, _T), :, :], i_v.at[b],
                              sem_i.at[b]).wait()

    def compute(b):
        for t in range(_T):
            for l in range(_LANES // 16):
                acc = (u_v[b, 0, t, 0, pl.ds(l * 16, 16)]
                       * i_v[b, 0, t, 0, pl.ds(l * 16, 16)])
                for tr in range(_TR):
                    for r in range(8):
                        if tr == 0 and r == 0:
                            continue
                        acc = acc + (u_v[b, tr, t, r, pl.ds(l * 16, 16)]
                                     * i_v[b, tr, t, r, pl.ds(l * 16, 16)])
                o_v[b, pl.ds(t * _LANES + l * 16, 16)] = acc

    def issue_out(si, b):
        off = out_w + si * _T * _LANES
        pltpu.async_copy(o_v.at[b], o_hbm.at[pl.ds(off, _T * _LANES)],
                         sem_o.at[b])

    def wait_out(b):
        pltpu.make_async_copy(o_v.at[b], o_hbm.at[pl.ds(0, _T * _LANES)],
                              sem_o.at[b]).wait()

    issue_in(0, 0)

    def grp(g, carry):
        si0 = g * 2
        issue_in(si0 + 1, 1)
        wait_in(0)

        @pl.when(si0 >= 2)
        def _():
            wait_out(0)

        compute(0)
        issue_out(si0, 0)

        @pl.when(si0 + 2 < _SC_STEPS)
        def _():
            issue_in(si0 + 2, 0)

        wait_in(1)

        @pl.when(si0 >= 1)
        def _():
            wait_out(1)

        compute(1)
        issue_out(si0 + 1, 1)
        return carry

    lax.fori_loop(0, _SC_STEPS // 2, grp, 0)
    wait_out(0)
    wait_out(1)


def kernel(gu, gi):
    gu = jnp.squeeze(gu)
    gi = jnp.squeeze(gi)
    n, k = gu.shape
    ut = gu.T
    it = gi.T
    nsc = _NSC
    ntc = n - nsc

    bn = 32768
    grid = pl.cdiv(ntc, bn)
    tc_out = pl.pallas_call(
        _tc_body,
        grid=(grid,),
        in_specs=[
            pl.BlockSpec((k, bn), lambda i: (0, i)),
            pl.BlockSpec((k, bn), lambda i: (0, i)),
        ],
        out_specs=pl.BlockSpec((bn,), lambda i: (i,)),
        out_shape=jax.ShapeDtypeStruct((ntc,), jnp.float32),
    )(ut, it)

    tcols = n // _LANES
    vt_u = ut.reshape(_TR, 8, tcols, _LANES).transpose(0, 2, 1, 3)
    vt_i = it.reshape(_TR, 8, tcols, _LANES).transpose(0, 2, 1, 3)
    sc_tc0 = ntc // _LANES

    mesh = plsc.VectorSubcoreMesh(core_axis_name="c", subcore_axis_name="s")
    sc_out = pl.kernel(
        functools.partial(_sc_body, sc_tc0),
        out_type=jax.ShapeDtypeStruct((nsc,), jnp.float32),
        mesh=mesh,
        scratch_types=[
            pltpu.VMEM((2, _TR, _T, 8, _LANES), jnp.float32),
            pltpu.VMEM((2, _TR, _T, 8, _LANES), jnp.float32),
            pltpu.VMEM((2, _T * _LANES), jnp.float32),
            pltpu.SemaphoreType.DMA((2,)),
            pltpu.SemaphoreType.DMA((2,)),
            pltpu.SemaphoreType.DMA((2,)),
        ],
    )(vt_u, vt_i)

    return jnp.concatenate([tc_out, sc_out])


# speedup vs baseline: 1.5441x; 1.0447x over previous
import jax
import jax.numpy as jnp
from jax import lax
from jax.experimental import pallas as pl
from jax.experimental.pallas import tpu as pltpu
from jax.experimental.pallas import tpu_sc as plsc
import functools

_K = 32
_TR = _K // 8
_LANES = 128
_T = 2
_SC_STEPS = 28
_NW = 32
_NSC = _NW * _SC_STEPS * _T * _LANES


def _tc_body(u_ref, i_ref, o_ref):
    o_ref[...] = jnp.sum(u_ref[...] * i_ref[...], axis=0)


def _sc_body(tc0, u_hbm, i_hbm, o_hbm, u_v, i_v, o_v, sem_u, sem_i, sem_o):
    cid = lax.axis_index("c")
    sid = lax.axis_index("s")
    wid = sid * 2 + cid
    tc_w = tc0 + wid * (_SC_STEPS * _T)
    out_w = wid * _SC_STEPS * _T * _LANES

    def issue_in(si, b):
        tcol = tc_w + si * _T
        pltpu.async_copy(u_hbm.at[:, pl.ds(tcol, _T), :, :], u_v.at[b],
                         sem_u.at[b])
        pltpu.async_copy(i_hbm.at[:, pl.ds(tcol, _T), :, :], i_v.at[b],
                         sem_i.at[b])

    def wait_in(b):
        pltpu.make_async_copy(u_hbm.at[:, pl.ds(0, _T), :, :], u_v.at[b],
                              sem_u.at[b]).wait()
        pltpu.make_async_copy(i_hbm.at[:, pl.ds(0, _T), :, :], i_v.at[b],
                              sem_i.at[b]).wait()

    def compute(b):
        for t in range(_T):
            for l in range(_LANES // 16):
                acc = (u_v[b, 0, t, 0, pl.ds(l * 16, 16)]
                       * i_v[b, 0, t, 0, pl.ds(l * 16, 16)])
                for tr in range(_TR):
                    for r in range(8):
                        if tr == 0 and r == 0:
                            continue
                        acc = acc + (u_v[b, tr, t, r, pl.ds(l * 16, 16)]
                                     * i_v[b, tr, t, r, pl.ds(l * 16, 16)])
                o_v[b, pl.ds(t * _LANES + l * 16, 16)] = acc

    def issue_out(si, b):
        off = out_w + si * _T * _LANES
        pltpu.async_copy(o_v.at[b], o_hbm.at[pl.ds(off, _T * _LANES)],
                         sem_o.at[b])

    def wait_out(b):
        pltpu.make_async_copy(o_v.at[b], o_hbm.at[pl.ds(0, _T * _LANES)],
                              sem_o.at[b]).wait()

    issue_in(0, 0)

    def grp(g, carry):
        si0 = g * 2
        issue_in(si0 + 1, 1)
        wait_in(0)

        @pl.when(si0 >= 2)
        def _():
            wait_out(0)

        compute(0)
        issue_out(si0, 0)

        @pl.when(si0 + 2 < _SC_STEPS)
        def _():
            issue_in(si0 + 2, 0)

        wait_in(1)

        @pl.when(si0 >= 1)
        def _():
            wait_out(1)

        compute(1)
        issue_out(si0 + 1, 1)
        return carry

    lax.fori_loop(0, _SC_STEPS // 2, grp, 0)
    wait_out(0)
    wait_out(1)


def kernel(gu, gi):
    gu = jnp.squeeze(gu)
    gi = jnp.squeeze(gi)
    n, k = gu.shape
    ut = gu.T
    it = gi.T
    nsc = _NSC
    ntc = n - nsc

    bn = 32768
    grid = pl.cdiv(ntc, bn)
    tc_out = pl.pallas_call(
        _tc_body,
        grid=(grid,),
        in_specs=[
            pl.BlockSpec((k, bn), lambda i: (0, i)),
            pl.BlockSpec((k, bn), lambda i: (0, i)),
        ],
        out_specs=pl.BlockSpec((bn,), lambda i: (i,)),
        out_shape=jax.ShapeDtypeStruct((ntc,), jnp.float32),
    )(ut, it)

    tcols = n // _LANES
    vt_u = ut.reshape(_TR, 8, tcols, _LANES).transpose(0, 2, 1, 3)
    vt_i = it.reshape(_TR, 8, tcols, _LANES).transpose(0, 2, 1, 3)
    sc_tc0 = ntc // _LANES

    mesh = plsc.VectorSubcoreMesh(core_axis_name="c", subcore_axis_name="s")
    sc_out = pl.kernel(
        functools.partial(_sc_body, sc_tc0),
        out_type=jax.ShapeDtypeStruct((nsc,), jnp.float32),
        mesh=mesh,
        scratch_types=[
            pltpu.VMEM((2, _TR, _T, 8, _LANES), jnp.float32),
            pltpu.VMEM((2, _TR, _T, 8, _LANES), jnp.float32),
            pltpu.VMEM((2, _T * _LANES), jnp.float32),
            pltpu.SemaphoreType.DMA((2,)),
            pltpu.SemaphoreType.DMA((2,)),
            pltpu.SemaphoreType.DMA((2,)),
        ],
    )(vt_u, vt_i)

    return jnp.concatenate([tc_out, sc_out])


# speedup vs baseline: 1.5445x; 1.0002x over previous
import jax
import jax.numpy as jnp
from jax import lax
from jax.experimental import pallas as pl
from jax.experimental.pallas import tpu as pltpu
from jax.experimental.pallas import tpu_sc as plsc
import functools

_K = 32
_TR = _K // 8
_LANES = 128
_T = 2
_SC_STEPS = 28
_NW = 32
_NSC = _NW * _SC_STEPS * _T * _LANES


def _tc_body(u_ref, i_ref, o_ref):
    o_ref[...] = jnp.sum(u_ref[...] * i_ref[...], axis=0)


def _sc_body(tc0, u_hbm, i_hbm, o_hbm, u_v, i_v, o_v, sem_u, sem_i, sem_o):
    cid = lax.axis_index("c")
    sid = lax.axis_index("s")
    wid = sid * 2 + cid

    def issue_in(si, b):
        tcol = tc0 + (si * _NW + wid) * _T
        pltpu.async_copy(u_hbm.at[:, pl.ds(tcol, _T), :, :], u_v.at[b],
                         sem_u.at[b])
        pltpu.async_copy(i_hbm.at[:, pl.ds(tcol, _T), :, :], i_v.at[b],
                         sem_i.at[b])

    def wait_in(b):
        pltpu.make_async_copy(u_hbm.at[:, pl.ds(0, _T), :, :], u_v.at[b],
                              sem_u.at[b]).wait()
        pltpu.make_async_copy(i_hbm.at[:, pl.ds(0, _T), :, :], i_v.at[b],
                              sem_i.at[b]).wait()

    def compute(b):
        for t in range(_T):
            for l in range(_LANES // 16):
                acc = (u_v[b, 0, t, 0, pl.ds(l * 16, 16)]
                       * i_v[b, 0, t, 0, pl.ds(l * 16, 16)])
                for tr in range(_TR):
                    for r in range(8):
                        if tr == 0 and r == 0:
                            continue
                        acc = acc + (u_v[b, tr, t, r, pl.ds(l * 16, 16)]
                                     * i_v[b, tr, t, r, pl.ds(l * 16, 16)])
                o_v[b, pl.ds(t * _LANES + l * 16, 16)] = acc

    def issue_out(si, b):
        off = (si * _NW + wid) * _T * _LANES
        pltpu.async_copy(o_v.at[b], o_hbm.at[pl.ds(off, _T * _LANES)],
                         sem_o.at[b])

    def wait_out(b):
        pltpu.make_async_copy(o_v.at[b], o_hbm.at[pl.ds(0, _T * _LANES)],
                              sem_o.at[b]).wait()

    issue_in(0, 0)

    def grp(g, carry):
        si0 = g * 2
        issue_in(si0 + 1, 1)
        wait_in(0)

        @pl.when(si0 >= 2)
        def _():
            wait_out(0)

        compute(0)
        issue_out(si0, 0)

        @pl.when(si0 + 2 < _SC_STEPS)
        def _():
            issue_in(si0 + 2, 0)

        wait_in(1)

        @pl.when(si0 >= 1)
        def _():
            wait_out(1)

        compute(1)
        issue_out(si0 + 1, 1)
        return carry

    lax.fori_loop(0, _SC_STEPS // 2, grp, 0)
    wait_out(0)
    wait_out(1)


def kernel(gu, gi):
    gu = jnp.squeeze(gu)
    gi = jnp.squeeze(gi)
    n, k = gu.shape
    ut = gu.T
    it = gi.T
    nsc = _NSC
    ntc = n - nsc

    bn = 32768
    grid = pl.cdiv(ntc, bn)
    tc_out = pl.pallas_call(
        _tc_body,
        grid=(grid,),
        in_specs=[
            pl.BlockSpec((k, bn), lambda i: (0, i)),
            pl.BlockSpec((k, bn), lambda i: (0, i)),
        ],
        out_specs=pl.BlockSpec((bn,), lambda i: (i,)),
        out_shape=jax.ShapeDtypeStruct((ntc,), jnp.float32),
    )(ut, it)

    tcols = n // _LANES
    vt_u = ut.reshape(_TR, 8, tcols, _LANES).transpose(0, 2, 1, 3)
    vt_i = it.reshape(_TR, 8, tcols, _LANES).transpose(0, 2, 1, 3)
    sc_tc0 = ntc // _LANES

    mesh = plsc.VectorSubcoreMesh(core_axis_name="c", subcore_axis_name="s")
    sc_out = pl.kernel(
        functools.partial(_sc_body, sc_tc0),
        out_type=jax.ShapeDtypeStruct((nsc,), jnp.float32),
        mesh=mesh,
        scratch_types=[
            pltpu.VMEM((2, _TR, _T, 8, _LANES), jnp.float32),
            pltpu.VMEM((2, _TR, _T, 8, _LANES), jnp.float32),
            pltpu.VMEM((2, _T * _LANES), jnp.float32),
            pltpu.SemaphoreType.DMA((2,)),
            pltpu.SemaphoreType.DMA((2,)),
            pltpu.SemaphoreType.DMA((2,)),
        ],
    )(vt_u, vt_i)

    return jnp.concatenate([tc_out, sc_out])


# speedup vs baseline: 1.5530x; 1.0055x over previous
import jax
import jax.numpy as jnp
from jax import lax
from jax.experimental import pallas as pl
from jax.experimental.pallas import tpu as pltpu
from jax.experimental.pallas import tpu_sc as plsc
import functools

_K = 32
_TR = _K // 8
_LANES = 128
_T = 2
_SC_STEPS = 2
_NW = 32
_NSC = _NW * _SC_STEPS * _T * _LANES


def _tc_body(u_ref, i_ref, o_ref):
    o_ref[...] = jnp.sum(u_ref[...] * i_ref[...], axis=0)


def _sc_body(tc0, u_hbm, i_hbm, o_hbm, u_v, i_v, o_v, sem_u, sem_i, sem_o):
    cid = lax.axis_index("c")
    sid = lax.axis_index("s")
    wid = sid * 2 + cid

    def issue_in(si, b):
        tcol = tc0 + (si * _NW + wid) * _T
        pltpu.async_copy(u_hbm.at[:, pl.ds(tcol, _T), :, :], u_v.at[b],
                         sem_u.at[b])
        pltpu.async_copy(i_hbm.at[:, pl.ds(tcol, _T), :, :], i_v.at[b],
                         sem_i.at[b])

    def wait_in(b):
        pltpu.make_async_copy(u_hbm.at[:, pl.ds(0, _T), :, :], u_v.at[b],
                              sem_u.at[b]).wait()
        pltpu.make_async_copy(i_hbm.at[:, pl.ds(0, _T), :, :], i_v.at[b],
                              sem_i.at[b]).wait()

    def compute(b):
        for t in range(_T):
            for l in range(_LANES // 16):
                acc = (u_v[b, 0, t, 0, pl.ds(l * 16, 16)]
                       * i_v[b, 0, t, 0, pl.ds(l * 16, 16)])
                for tr in range(_TR):
                    for r in range(8):
                        if tr == 0 and r == 0:
                            continue
                        acc = acc + (u_v[b, tr, t, r, pl.ds(l * 16, 16)]
                                     * i_v[b, tr, t, r, pl.ds(l * 16, 16)])
                o_v[b, pl.ds(t * _LANES + l * 16, 16)] = acc

    def issue_out(si, b):
        off = (si * _NW + wid) * _T * _LANES
        pltpu.async_copy(o_v.at[b], o_hbm.at[pl.ds(off, _T * _LANES)],
                         sem_o.at[b])

    def wait_out(b):
        pltpu.make_async_copy(o_v.at[b], o_hbm.at[pl.ds(0, _T * _LANES)],
                              sem_o.at[b]).wait()

    issue_in(0, 0)

    def grp(g, carry):
        si0 = g * 2
        issue_in(si0 + 1, 1)
        wait_in(0)

        @pl.when(si0 >= 2)
        def _():
            wait_out(0)

        compute(0)
        issue_out(si0, 0)

        @pl.when(si0 + 2 < _SC_STEPS)
        def _():
            issue_in(si0 + 2, 0)

        wait_in(1)

        @pl.when(si0 >= 1)
        def _():
            wait_out(1)

        compute(1)
        issue_out(si0 + 1, 1)
        return carry

    lax.fori_loop(0, _SC_STEPS // 2, grp, 0)
    wait_out(0)
    wait_out(1)


def kernel(gu, gi):
    gu = jnp.squeeze(gu)
    gi = jnp.squeeze(gi)
    n, k = gu.shape
    ut = gu.T
    it = gi.T
    nsc = _NSC
    ntc = n - nsc

    bn = 32768
    grid = pl.cdiv(ntc, bn)
    tc_out = pl.pallas_call(
        _tc_body,
        grid=(grid,),
        in_specs=[
            pl.BlockSpec((k, bn), lambda i: (0, i)),
            pl.BlockSpec((k, bn), lambda i: (0, i)),
        ],
        out_specs=pl.BlockSpec((bn,), lambda i: (i,)),
        out_shape=jax.ShapeDtypeStruct((ntc,), jnp.float32),
    )(ut, it)

    tcols = n // _LANES
    vt_u = ut.reshape(_TR, 8, tcols, _LANES).transpose(0, 2, 1, 3)
    vt_i = it.reshape(_TR, 8, tcols, _LANES).transpose(0, 2, 1, 3)
    sc_tc0 = ntc // _LANES

    mesh = plsc.VectorSubcoreMesh(core_axis_name="c", subcore_axis_name="s")
    sc_out = pl.kernel(
        functools.partial(_sc_body, sc_tc0),
        out_type=jax.ShapeDtypeStruct((nsc,), jnp.float32),
        mesh=mesh,
        scratch_types=[
            pltpu.VMEM((2, _TR, _T, 8, _LANES), jnp.float32),
            pltpu.VMEM((2, _TR, _T, 8, _LANES), jnp.float32),
            pltpu.VMEM((2, _T * _LANES), jnp.float32),
            pltpu.SemaphoreType.DMA((2,)),
            pltpu.SemaphoreType.DMA((2,)),
            pltpu.SemaphoreType.DMA((2,)),
        ],
    )(vt_u, vt_i)

    return jnp.concatenate([tc_out, sc_out])


# speedup vs baseline: 1.8435x; 1.1871x over previous
import jax
import jax.numpy as jnp
from jax.experimental import pallas as pl


def _body(u_ref, i_ref, o_ref):
    o_ref[...] = jnp.sum(u_ref[...] * i_ref[...], axis=0)


def kernel(gu, gi):
    gu = jnp.squeeze(gu)
    gi = jnp.squeeze(gi)
    n, k = gu.shape
    ut = gu.T
    it = gi.T
    bn = 32768
    grid = pl.cdiv(n, bn)
    return pl.pallas_call(
        _body,
        grid=(grid,),
        in_specs=[
            pl.BlockSpec((k, bn), lambda i: (0, i)),
            pl.BlockSpec((k, bn), lambda i: (0, i)),
        ],
        out_specs=pl.BlockSpec((bn,), lambda i: (i,)),
        out_shape=jax.ShapeDtypeStruct((n,), jnp.float32),
    )(ut, it)
